# baseline (speedup 1.0000x reference)
import functools

import jax
import jax.numpy as jnp
import numpy as np
from jax import lax
from jax.experimental import pallas as pl
from jax.experimental.pallas import tpu as pltpu
from jax.experimental.pallas import tpu_sc as plsc

B = 16384
IN_DIM = 169
IN_PAD = 256
HIDDEN = 2048
ACTIONS = 1024
N_BANNED = 4096
MIN_VAL = float(np.finfo(np.float32).min)

TILE = 1024
GRID = B // TILE

NUM_CORES = 2
NUM_SUBCORES = 16
NW = NUM_CORES * NUM_SUBCORES
ROWS_PER_W = B // NW
LANES = 16
COLS_PER_WORD = 8
WORDS_PER_ROW = ACTIONS // COLS_PER_WORD
SLAB = ROWS_PER_W * WORDS_PER_ROW
POS_PER_W = ROWS_PER_W * ACTIONS


@functools.lru_cache(maxsize=1)
def _make_mask_kernel():
    mesh = plsc.VectorSubcoreMesh(core_axis_name="c", subcore_axis_name="s")

    zu = 16

    @functools.partial(
        pl.kernel,
        mesh=mesh,
        out_type=jax.ShapeDtypeStruct((B * WORDS_PER_ROW,), jnp.int32),
        scratch_types=[
            pltpu.VMEM((N_BANNED,), jnp.int32),
            pltpu.VMEM((SLAB,), jnp.int32),
        ],
        compiler_params=pltpu.CompilerParams(needs_layout_passes=False),
    )
    def mask_kernel(banned_hbm, bits_hbm, banned_v, slab_v):
        wid = lax.axis_index("s") * NUM_CORES + lax.axis_index("c")
        pltpu.sync_copy(banned_hbm, banned_v)
        zeros16 = jnp.zeros((LANES,), jnp.int32)
        base = wid * POS_PER_W

        def zbody(i, c):
            for u in range(zu):
                slab_v[pl.ds((i * zu + u) * LANES, LANES)] = zeros16
            return c

        lax.fori_loop(0, SLAB // (LANES * zu), zbody, 0)

        def sbody(j, c):
            idx = banned_v[pl.ds(j * LANES, LANES)]
            local = idx - base
            ok = (local >= 0) & (local < POS_PER_W)
            word = jnp.clip(local >> 3, 0, SLAB - 1)
            bitpos = local & 7
            for bb in range(COLS_PER_WORD):
                m = ok & (bitpos == bb)
                old = plsc.load_gather(slab_v, [word], mask=m)
                plsc.store_scatter(slab_v, [word],
                                   old | jnp.int32(1 << bb), mask=m)
            return c

        lax.fori_loop(0, N_BANNED // LANES, sbody, 0)
        pltpu.sync_copy(slab_v, bits_hbm.at[pl.ds(wid * SLAB, SLAB)])

    return mask_kernel


def _mlp_body(xt_ref, w1_ref, b1_ref, w2_ref, b2_ref, raw_ref,
              w1_bf, w2_bf):
    @pl.when(pl.program_id(0) == 0)
    def _():
        w1_bf[...] = w1_ref[...].astype(jnp.bfloat16)
        w2_bf[...] = w2_ref[...].astype(jnp.bfloat16)

    h = jnp.maximum(
        lax.dot_general(xt_ref[...].astype(jnp.bfloat16), w1_bf[...],
                        dimension_numbers=(((0,), (0,)), ((), ())),
                        preferred_element_type=jnp.float32)
        + b1_ref[...], 0.0)
    raw_ref[...] = (
        jnp.dot(h.astype(jnp.bfloat16), w2_bf[...],
                preferred_element_type=jnp.float32)
        + b2_ref[...])


_mlp_call = pl.pallas_call(
    _mlp_body,
    grid=(GRID,),
    in_specs=[
        pl.BlockSpec((IN_DIM, TILE), lambda i: (0, i)),
        pl.BlockSpec((IN_DIM, HIDDEN), lambda i: (0, 0)),
        pl.BlockSpec((1, HIDDEN), lambda i: (0, 0)),
        pl.BlockSpec((HIDDEN, ACTIONS), lambda i: (0, 0)),
        pl.BlockSpec((1, ACTIONS), lambda i: (0, 0)),
    ],
    out_specs=pl.BlockSpec((TILE, ACTIONS), lambda i: (i, 0)),
    out_shape=jax.ShapeDtypeStruct((B, ACTIONS), jnp.float32),
    scratch_shapes=[
        pltpu.VMEM((IN_DIM, HIDDEN), jnp.bfloat16),
        pltpu.VMEM((HIDDEN, ACTIONS), jnp.bfloat16),
    ],
    compiler_params=pltpu.CompilerParams(
        dimension_semantics=("arbitrary",)),
)

TILE2 = 1024
GRID2 = B // TILE2


def _top1_body(raw_ref, bits_ref, idx_ref, val_ref):
    raw = raw_ref[...]
    bits = bits_ref[...].reshape(TILE2, WORDS_PER_ROW)
    wrow = lax.broadcasted_iota(jnp.int32, (WORDS_PER_ROW, ACTIONS), 0)
    ccol = lax.broadcasted_iota(jnp.int32, (WORDS_PER_ROW, ACTIONS), 1)
    sel = ((ccol >> 3) == wrow).astype(jnp.bfloat16)
    rep = jnp.dot(bits.astype(jnp.bfloat16), sel,
                  preferred_element_type=jnp.float32).astype(jnp.int32)
    col = lax.broadcasted_iota(jnp.int32, raw.shape, 1)
    banned = (lax.shift_right_logical(rep, col & 7) & 1) != 0
    q = jnp.where(banned, MIN_VAL, raw)
    vmax = jnp.max(q, axis=1, keepdims=True)
    idx = jnp.min(jnp.where(q == vmax, col, jnp.int32(ACTIONS)),
                  axis=1, keepdims=True)
    idx_ref[...] = idx.reshape(TILE2 // 128, 128)
    val_ref[...] = vmax.reshape(TILE2 // 128, 128)


_top1_call = pl.pallas_call(
    _top1_body,
    grid=(GRID2,),
    in_specs=[
        pl.BlockSpec((TILE2, ACTIONS), lambda i: (i, 0)),
        pl.BlockSpec((TILE2 // 8, 8, WORDS_PER_ROW), lambda i: (i, 0, 0)),
    ],
    out_specs=[
        pl.BlockSpec((TILE2 // 128, 128), lambda i: (i, 0)),
        pl.BlockSpec((TILE2 // 128, 128), lambda i: (i, 0)),
    ],
    out_shape=[
        jax.ShapeDtypeStruct((B // 128, 128), jnp.int32),
        jax.ShapeDtypeStruct((B // 128, 128), jnp.float32),
    ],
    compiler_params=pltpu.CompilerParams(
        dimension_semantics=("arbitrary",)),
)


def kernel(encoded_graphs, banned_acts, W1, b1, W2, b2):
    bits = _make_mask_kernel()(banned_acts.astype(jnp.int32))
    bits3 = bits.reshape(B // 8, 8, WORDS_PER_ROW)
    raw = _mlp_call(encoded_graphs.T, W1, b1.reshape(1, HIDDEN), W2,
                    b2.reshape(1, ACTIONS))
    idx, val = _top1_call(raw, bits3)
    return (idx.reshape(B, 1), val.reshape(B, 1), raw)


# speedup vs baseline: 1.0226x; 1.0226x over previous
import functools

import jax
import jax.numpy as jnp
import numpy as np
from jax import lax
from jax.experimental import pallas as pl
from jax.experimental.pallas import tpu as pltpu
from jax.experimental.pallas import tpu_sc as plsc

B = 16384
IN_DIM = 169
IN_PAD = 256
HIDDEN = 2048
ACTIONS = 1024
N_BANNED = 4096
MIN_VAL = float(np.finfo(np.float32).min)

TILE = 1024
GRID = B // TILE

NUM_CORES = 2
NUM_SUBCORES = 16
NW = NUM_CORES * NUM_SUBCORES
ROWS_PER_W = B // NW
LANES = 16
COLS_PER_WORD = 8
WORDS_PER_ROW = ACTIONS // COLS_PER_WORD
SLAB = ROWS_PER_W * WORDS_PER_ROW
POS_PER_W = ROWS_PER_W * ACTIONS


@functools.lru_cache(maxsize=1)
def _make_mask_kernel():
    mesh = plsc.VectorSubcoreMesh(core_axis_name="c", subcore_axis_name="s")

    zu = 16

    @functools.partial(
        pl.kernel,
        mesh=mesh,
        out_type=jax.ShapeDtypeStruct((B * WORDS_PER_ROW,), jnp.int32),
        scratch_types=[
            pltpu.VMEM((N_BANNED,), jnp.int32),
            pltpu.VMEM((SLAB,), jnp.int32),
        ],
        compiler_params=pltpu.CompilerParams(needs_layout_passes=False),
    )
    def mask_kernel(banned_hbm, bits_hbm, banned_v, slab_v):
        wid = lax.axis_index("s") * NUM_CORES + lax.axis_index("c")
        pltpu.sync_copy(banned_hbm, banned_v)
        zeros16 = jnp.zeros((LANES,), jnp.int32)
        base = wid * POS_PER_W

        def zbody(i, c):
            for u in range(zu):
                slab_v[pl.ds((i * zu + u) * LANES, LANES)] = zeros16
            return c

        lax.fori_loop(0, SLAB // (LANES * zu), zbody, 0)

        def sbody(j, c):
            idx = banned_v[pl.ds(j * LANES, LANES)]
            local = idx - base
            ok = (local >= 0) & (local < POS_PER_W)
            word = jnp.clip(local >> 3, 0, SLAB - 1)
            bitpos = local & 7
            for bb in range(COLS_PER_WORD):
                m = ok & (bitpos == bb)
                old = plsc.load_gather(slab_v, [word], mask=m)
                plsc.store_scatter(slab_v, [word],
                                   old | jnp.int32(1 << bb), mask=m)
            return c

        lax.fori_loop(0, N_BANNED // LANES, sbody, 0)
        pltpu.sync_copy(slab_v, bits_hbm.at[pl.ds(wid * SLAB, SLAB)])

    return mask_kernel


def _mlp_body(xt_ref, w1_ref, b1_ref, w2_ref, b2_ref, raw_ref,
              w1_bf, w2_bf):
    @pl.when(pl.program_id(0) == 0)
    def _():
        w1_bf[...] = w1_ref[...].astype(jnp.bfloat16)
        w2_bf[...] = w2_ref[...].astype(jnp.bfloat16)

    h = jnp.maximum(
        lax.dot_general(xt_ref[...].astype(jnp.bfloat16), w1_bf[...],
                        dimension_numbers=(((0,), (0,)), ((), ())),
                        preferred_element_type=jnp.float32)
        + b1_ref[...], 0.0)
    raw_ref[...] = (
        jnp.dot(h.astype(jnp.bfloat16), w2_bf[...],
                preferred_element_type=jnp.float32)
        + b2_ref[...])


_mlp_call = pl.pallas_call(
    _mlp_body,
    grid=(GRID,),
    in_specs=[
        pl.BlockSpec((IN_DIM, TILE), lambda i: (0, i)),
        pl.BlockSpec((IN_DIM, HIDDEN), lambda i: (0, 0)),
        pl.BlockSpec((1, HIDDEN), lambda i: (0, 0)),
        pl.BlockSpec((HIDDEN, ACTIONS), lambda i: (0, 0)),
        pl.BlockSpec((1, ACTIONS), lambda i: (0, 0)),
    ],
    out_specs=pl.BlockSpec((TILE, ACTIONS), lambda i: (i, 0)),
    out_shape=jax.ShapeDtypeStruct((B, ACTIONS), jnp.float32),
    scratch_shapes=[
        pltpu.VMEM((IN_DIM, HIDDEN), jnp.bfloat16),
        pltpu.VMEM((HIDDEN, ACTIONS), jnp.bfloat16),
    ],
    compiler_params=pltpu.CompilerParams(
        dimension_semantics=("arbitrary",)),
)

TILE2 = 2048
GRID2 = B // TILE2


def _top1_body(raw_ref, bits_ref, idx_ref, val_ref):
    raw = raw_ref[...]
    bits = bits_ref[...].reshape(TILE2, WORDS_PER_ROW)
    wrow = lax.broadcasted_iota(jnp.int32, (WORDS_PER_ROW, ACTIONS), 0)
    ccol = lax.broadcasted_iota(jnp.int32, (WORDS_PER_ROW, ACTIONS), 1)
    sel = ((ccol >> 3) == wrow).astype(jnp.bfloat16)
    rep = jnp.dot(bits.astype(jnp.bfloat16), sel,
                  preferred_element_type=jnp.float32).astype(jnp.int32)
    col = lax.broadcasted_iota(jnp.int32, raw.shape, 1)
    banned = (lax.shift_right_logical(rep, col & 7) & 1) != 0
    q = jnp.where(banned, MIN_VAL, raw)
    vmax = jnp.max(q, axis=1, keepdims=True)
    idx = jnp.min(jnp.where(q == vmax, col, jnp.int32(ACTIONS)),
                  axis=1, keepdims=True)
    idx_ref[...] = idx.reshape(TILE2 // 128, 128)
    val_ref[...] = vmax.reshape(TILE2 // 128, 128)


_top1_call = pl.pallas_call(
    _top1_body,
    grid=(GRID2,),
    in_specs=[
        pl.BlockSpec((TILE2, ACTIONS), lambda i: (i, 0)),
        pl.BlockSpec((TILE2 // 8, 8, WORDS_PER_ROW), lambda i: (i, 0, 0)),
    ],
    out_specs=[
        pl.BlockSpec((TILE2 // 128, 128), lambda i: (i, 0)),
        pl.BlockSpec((TILE2 // 128, 128), lambda i: (i, 0)),
    ],
    out_shape=[
        jax.ShapeDtypeStruct((B // 128, 128), jnp.int32),
        jax.ShapeDtypeStruct((B // 128, 128), jnp.float32),
    ],
    compiler_params=pltpu.CompilerParams(
        dimension_semantics=("arbitrary",)),
)


def kernel(encoded_graphs, banned_acts, W1, b1, W2, b2):
    bits = _make_mask_kernel()(banned_acts.astype(jnp.int32))
    bits3 = bits.reshape(B // 8, 8, WORDS_PER_ROW)
    raw = _mlp_call(encoded_graphs.T, W1, b1.reshape(1, HIDDEN), W2,
                    b2.reshape(1, ACTIONS))
    idx, val = _top1_call(raw, bits3)
    return (idx.reshape(B, 1), val.reshape(B, 1), raw)
